# Initial kernel scaffold; baseline (speedup 1.0000x reference)
import jax
import jax.numpy as jnp
from jax.experimental import pallas as pl


def kernel(x, weight):
    raise NotImplementedError("write your pallas kernel here")


# speedup vs baseline: 2.4292x; 2.4292x over previous
import functools

import jax
import jax.numpy as jnp
from jax import lax
from jax.experimental import pallas as pl
from jax.experimental.pallas import tpu as pltpu
from jax.experimental.pallas import tpu_sc as plsc

NUM_EMB = 100000
DIM = 128
B_TOTAL = 4096 * 50
NC, NS = 2, 16
NW = NC * NS
ROWS_PER_CHUNK = 128
CPW = B_TOTAL // (NW * ROWS_PER_CHUNK)


def _rsqrt_nr(x):
    i = lax.bitcast_convert_type(x, jnp.int32)
    i = jnp.int32(0x5F3759DF) - lax.shift_right_arithmetic(i, 1)
    y = lax.bitcast_convert_type(i, jnp.float32)
    for _ in range(3):
        y = y * (1.5 - 0.5 * x * y * y)
    return y


_mesh = plsc.VectorSubcoreMesh(core_axis_name="c", subcore_axis_name="s")


@functools.partial(
    pl.kernel,
    mesh=_mesh,
    out_type=jax.ShapeDtypeStruct((B_TOTAL, DIM), jnp.float32),
    scratch_types=[
        pltpu.VMEM((1, CPW, ROWS_PER_CHUNK), jnp.int32),
        pltpu.VMEM((ROWS_PER_CHUNK, DIM), jnp.float32),
        pltpu.SemaphoreType.DMA,
    ],
    compiler_params=pltpu.CompilerParams(needs_layout_passes=False),
)
def _embed(x_hbm, tab_hbm, out_hbm, idx_v, rows_v, sem):
    wid = lax.axis_index("s") * NC + lax.axis_index("c")
    pltpu.sync_copy(x_hbm.at[pl.ds(wid, 1)], idx_v)

    def _norm_row(r):
        vs = [rows_v[r, pl.ds(k * 16, 16)] for k in range(DIM // 16)]
        acc = vs[0] * vs[0]
        for v in vs[1:]:
            acc = acc + v * v
        s = jnp.sum(acc)
        scale = _rsqrt_nr(s * (1.0 / DIM) + 1e-4)
        for k, v in enumerate(vs):
            rows_v[r, pl.ds(k * 16, 16)] = v * scale

    UNROLL = 4

    def chunk(ci, carry):
        pltpu.async_copy(tab_hbm.at[idx_v.at[0, ci]], rows_v, sem).wait()

        def rows_body(i, c):
            for u in range(UNROLL):
                _norm_row(i * UNROLL + u)
            return c

        lax.fori_loop(0, ROWS_PER_CHUNK // UNROLL, rows_body, 0)
        pltpu.sync_copy(
            rows_v,
            out_hbm.at[pl.ds((wid * CPW + ci) * ROWS_PER_CHUNK, ROWS_PER_CHUNK)],
        )
        return carry

    lax.fori_loop(0, CPW, chunk, 0)


def kernel(x, weight):
    x2 = x.astype(jnp.int32).reshape(NW, CPW, 128)
    out = _embed(x2, weight)
    return out.reshape(4096, 50, DIM)
